# baseline (speedup 1.0000x reference)
import jax
import jax.numpy as jnp
from jax import lax
from jax.experimental import pallas as pl
from jax.experimental.pallas import tpu as pltpu
from jax.experimental.pallas import tpu_sc as plsc

DIM = 1024
ROWS = 8192
NC, NS, L = 2, 16, 16
NW = NC * NS
ROWS_PER_W = ROWS // NW
CHUNK = 32
NBUF = 3
N_CHUNKS = ROWS_PER_W // CHUNK
VECS_PER_ROW = DIM // L


def _sc_body(emb_hbm, out_hbm, buf, sems_in, sems_out, *, scale):
    wid = lax.axis_index("s") * NC + lax.axis_index("c")
    base = wid * ROWS_PER_W

    def start_in(g, slot):
        pltpu.async_copy(
            emb_hbm.at[pl.ds(base + g * CHUNK, CHUNK)], buf.at[slot], sems_in[slot]
        )

    def start_out(g, slot):
        pltpu.async_copy(
            buf.at[slot], out_hbm.at[pl.ds(base + g * CHUNK, CHUNK)], sems_out[slot]
        )

    def wait_in(slot):
        pltpu.make_async_copy(
            emb_hbm.at[pl.ds(0, CHUNK)], buf.at[slot], sems_in[slot]
        ).wait()

    def wait_out(slot):
        pltpu.make_async_copy(
            buf.at[slot], out_hbm.at[pl.ds(0, CHUNK)], sems_out[slot]
        ).wait()

    def compute_half(slot, h):
        def row_body(r, carry):
            for c in range(VECS_PER_ROW):
                v = buf[slot, r, pl.ds(c * L, L)]
                buf[slot, r, pl.ds(c * L, L)] = v * scale
            return carry

        lax.fori_loop(h * (CHUNK // 2), (h + 1) * (CHUNK // 2), row_body, jnp.int32(0))

    def start_out_half(g, slot, h):
        r0 = h * (CHUNK // 2)
        pltpu.async_copy(
            buf.at[slot].at[pl.ds(r0, CHUNK // 2)],
            out_hbm.at[pl.ds(base + g * CHUNK + r0, CHUNK // 2)],
            sems_out[slot],
        )

    start_in(0, 0)
    start_in(1, 1)
    for g in range(N_CHUNKS):
        slot = g % NBUF
        wait_in(slot)
        nxt = g + 2
        if nxt < N_CHUNKS:
            nslot = nxt % NBUF
            if nxt >= NBUF:
                wait_out(nslot)
            start_in(nxt, nslot)
        compute_half(slot, 0)
        start_out_half(g, slot, 0)
        compute_half(slot, 1)
        start_out_half(g, slot, 1)
    for g in range(N_CHUNKS - NBUF + 1, N_CHUNKS):
        wait_out(g % NBUF)


@jax.jit
def _sc_scaled_copy(emb):
    scale = DIM ** (-0.5)
    mesh = plsc.VectorSubcoreMesh(
        core_axis_name="c", subcore_axis_name="s", num_cores=NC, num_subcores=NS
    )

    def body(emb_hbm, out_hbm, buf, *sems):
        _sc_body(
            emb_hbm,
            out_hbm,
            buf,
            list(sems[:NBUF]),
            list(sems[NBUF:]),
            scale=scale,
        )

    return pl.kernel(
        body,
        out_type=jax.ShapeDtypeStruct((ROWS, DIM), jnp.float32),
        mesh=mesh,
        scratch_types=[pltpu.VMEM((NBUF, CHUNK, DIM), jnp.float32)]
        + [pltpu.SemaphoreType.DMA] * (2 * NBUF),
    )(emb)


def kernel(x, emb):
    del x
    return _sc_scaled_copy(emb)


# speedup vs baseline: 1.0491x; 1.0491x over previous
import jax
import jax.numpy as jnp
from jax import lax
from jax.experimental import pallas as pl
from jax.experimental.pallas import tpu as pltpu
from jax.experimental.pallas import tpu_sc as plsc

DIM = 1024
ROWS = 8192
NC, NS, L = 2, 16, 16
NW = NC * NS
ROWS_PER_W = ROWS // NW
CHUNK = 32
NBUF = 3
N_CHUNKS = ROWS_PER_W // CHUNK
VECS_PER_ROW = DIM // L


def _sc_body(emb_hbm, out_hbm, buf, sems_in, sems_out, *, scale):
    wid = lax.axis_index("s") * NC + lax.axis_index("c")
    base = wid * ROWS_PER_W

    def start_in(g, slot):
        pltpu.async_copy(
            emb_hbm.at[pl.ds(base + g * CHUNK, CHUNK)], buf.at[slot], sems_in[slot]
        )

    def start_out(g, slot):
        pltpu.async_copy(
            buf.at[slot], out_hbm.at[pl.ds(base + g * CHUNK, CHUNK)], sems_out[slot]
        )

    def wait_in(slot):
        pltpu.make_async_copy(
            emb_hbm.at[pl.ds(0, CHUNK)], buf.at[slot], sems_in[slot]
        ).wait()

    def wait_out(slot):
        pltpu.make_async_copy(
            buf.at[slot], out_hbm.at[pl.ds(0, CHUNK)], sems_out[slot]
        ).wait()

    def compute(slot):
        def row_body(r, carry):
            for c in range(VECS_PER_ROW):
                v = buf[slot, r, pl.ds(c * L, L)]
                buf[slot, r, pl.ds(c * L, L)] = v * scale
            return carry

        lax.fori_loop(0, CHUNK, row_body, jnp.int32(0))

    start_in(0, 0)
    start_in(1, 1)
    for g in range(N_CHUNKS):
        slot = g % NBUF
        wait_in(slot)
        nxt = g + 2
        if nxt < N_CHUNKS:
            nslot = nxt % NBUF
            if nxt >= NBUF:
                wait_out(nslot)
            start_in(nxt, nslot)
        compute(slot)
        start_out(g, slot)
    for g in range(N_CHUNKS - NBUF + 1, N_CHUNKS):
        wait_out(g % NBUF)


@jax.jit
def _sc_scaled_copy(emb):
    scale = DIM ** (-0.5)
    mesh = plsc.VectorSubcoreMesh(
        core_axis_name="c", subcore_axis_name="s", num_cores=NC, num_subcores=NS
    )

    def body(emb_hbm, out_hbm, buf, *sems):
        _sc_body(
            emb_hbm,
            out_hbm,
            buf,
            list(sems[:NBUF]),
            list(sems[NBUF:]),
            scale=scale,
        )

    return pl.kernel(
        body,
        out_type=jax.ShapeDtypeStruct((ROWS, DIM), jnp.float32),
        mesh=mesh,
        scratch_types=[pltpu.VMEM((NBUF, CHUNK, DIM), jnp.float32)]
        + [pltpu.SemaphoreType.DMA] * (2 * NBUF),
    )(emb)


def kernel(x, emb):
    del x
    return _sc_scaled_copy(emb)
